# Initial kernel scaffold; baseline (speedup 1.0000x reference)
import jax
import jax.numpy as jnp
from jax.experimental import pallas as pl


def kernel(x, edge_index, batch, node_imp, W1_0, b1_0, W2_0, b2_0, W1_1, b1_1, W2_1, b2_1, W1_2, b1_2, W2_2, b2_2):
    raise NotImplementedError("write your pallas kernel here")


# speedup vs baseline: 4.4484x; 4.4484x over previous
import functools

import jax
import jax.numpy as jnp
from jax import lax
from jax.experimental import pallas as pl
from jax.experimental.pallas import tpu as pltpu
from jax.experimental.pallas import tpu_sc as plsc

N = 10000
E = 320000
F = 128
G = 64
EPS = 1e-10
SCALAR = 20

NC = 2
NS = 16
NW = NC * NS
EPW = E // NW
CHUNK = 80
NCHUNK = EPW // CHUNK
ZR = 632
NP = NS * ZR

_mesh = plsc.VectorSubcoreMesh(core_axis_name="c", subcore_axis_name="s")


@functools.partial(
    pl.kernel,
    out_type=jax.ShapeDtypeStruct((NC, NP, F), jnp.float32),
    mesh=_mesh,
    scratch_types=[
        pltpu.VMEM((CHUNK,), jnp.int32),
        pltpu.VMEM((CHUNK,), jnp.int32),
        pltpu.VMEM((CHUNK, F), jnp.float32),
        pltpu.VMEM_SHARED((NP, F), jnp.float32),
        pltpu.SemaphoreType.DMA,
    ],
)
def _seg_sum(h_hbm, src_hbm, dst_hbm, zeros_hbm, out_hbm,
             src_v, dst_v, rows_v, agg_sh, sem):
    c = lax.axis_index("c")
    s = lax.axis_index("s")
    wid = c * NS + s

    pltpu.sync_copy(zeros_hbm, agg_sh.at[pl.ds(s * ZR, ZR)])
    plsc.subcore_barrier()

    def body(i, carry):
        base = wid * EPW + i * CHUNK
        pltpu.sync_copy(src_hbm.at[pl.ds(base, CHUNK)], src_v)
        pltpu.sync_copy(dst_hbm.at[pl.ds(base, CHUNK)], dst_v)
        pltpu.async_copy(h_hbm.at[src_v], rows_v, sem).wait()
        pltpu.sync_copy(rows_v, agg_sh.at[dst_v], add=True)
        return carry

    lax.fori_loop(0, NCHUNK, body, 0)
    plsc.subcore_barrier()
    pltpu.sync_copy(agg_sh.at[pl.ds(s * ZR, ZR)],
                    out_hbm.at[c, pl.ds(s * ZR, ZR)])


def _prescale_body(x_ref, batch_ref, nimp_ref, h0_ref):
    b = batch_ref[...]
    nimp = nimp_ref[...]
    gids = lax.broadcasted_iota(jnp.int32, (N, G), 1)
    mask = b == gids
    neg = jnp.float32(-1e30)
    segmax = jnp.max(jnp.where(mask, nimp, neg), axis=0, keepdims=True)
    pmax = jnp.max(jnp.where(mask, segmax, neg), axis=1, keepdims=True)
    ni = nimp / (pmax + EPS)
    ni = (2.0 * ni - 1.0) / (2.0 * SCALAR) + 1.0
    h0_ref[...] = x_ref[...] * ni


def _prescale(x, batch2d, nimp2d):
    return pl.pallas_call(
        _prescale_body,
        out_shape=jax.ShapeDtypeStruct((N, F), jnp.float32),
    )(x, batch2d, nimp2d)


_RB = 1000


def _layer_body(p_ref, h_ref, W1_ref, b1_ref, W2_ref, b2_ref, o_ref):
    z = p_ref[0] + p_ref[1] + h_ref[...]
    a = jnp.dot(z, W1_ref[...], preferred_element_type=jnp.float32)
    a = jnp.maximum(a + b1_ref[...], 0.0)
    o = jnp.dot(a, W2_ref[...], preferred_element_type=jnp.float32)
    o_ref[...] = jnp.maximum(o + b2_ref[...], 0.0)


def _gin_layer(p, h, W1, b1, W2, b2):
    nblk = N // _RB
    return pl.pallas_call(
        _layer_body,
        grid=(nblk,),
        in_specs=[
            pl.BlockSpec((NC, _RB, F), lambda i: (0, i, 0)),
            pl.BlockSpec((_RB, F), lambda i: (i, 0)),
            pl.BlockSpec((F, F), lambda i: (0, 0)),
            pl.BlockSpec((1, F), lambda i: (0, 0)),
            pl.BlockSpec((F, F), lambda i: (0, 0)),
            pl.BlockSpec((1, F), lambda i: (0, 0)),
        ],
        out_specs=pl.BlockSpec((_RB, F), lambda i: (i, 0)),
        out_shape=jax.ShapeDtypeStruct((N, F), jnp.float32),
    )(p, h, W1, b1, W2, b2)


def _pool_body(x1_ref, x2_ref, x3_ref, batch_ref, o_ref):
    b = batch_ref[...]
    gids = lax.broadcasted_iota(jnp.int32, (N, G), 1)
    mask = (b == gids).astype(jnp.float32)
    counts = jnp.maximum(jnp.sum(mask, axis=0, keepdims=True), 1.0)
    dn = (((0,), (0,)), ((), ()))
    for k, xr in enumerate((x1_ref, x2_ref, x3_ref)):
        p = lax.dot_general(mask, xr[...], dn,
                            preferred_element_type=jnp.float32)
        o_ref[:, k * F:(k + 1) * F] = p / counts.T


def _pool(x1, x2, x3, batch2d):
    return pl.pallas_call(
        _pool_body,
        out_shape=jax.ShapeDtypeStruct((G, 3 * F), jnp.float32),
    )(x1, x2, x3, batch2d)


def kernel(x, edge_index, batch, node_imp,
           W1_0, b1_0, W2_0, b2_0,
           W1_1, b1_1, W2_1, b2_1,
           W1_2, b1_2, W2_2, b2_2):
    src = edge_index[0].astype(jnp.int32)
    dst = edge_index[1].astype(jnp.int32)
    batch2d = batch.astype(jnp.int32).reshape(N, 1)
    nimp2d = node_imp.reshape(N, 1)
    zeros = jnp.zeros((ZR, F), jnp.float32)

    h = _prescale(x, batch2d, nimp2d)

    params = [(W1_0, b1_0, W2_0, b2_0),
              (W1_1, b1_1, W2_1, b2_1),
              (W1_2, b1_2, W2_2, b2_2)]
    xs = []
    for (W1, b1, W2, b2) in params:
        p = _seg_sum(h, src, dst, zeros)[:, :N]
        h = _gin_layer(p, h, W1, b1.reshape(1, F), W2, b2.reshape(1, F))
        xs.append(h)

    graph_emb = _pool(xs[0], xs[1], xs[2], batch2d)
    node_emb = jnp.concatenate(xs, axis=1)
    return graph_emb, node_emb


# speedup vs baseline: 9.5912x; 2.1561x over previous
import functools

import jax
import jax.numpy as jnp
from jax import lax
from jax.experimental import pallas as pl
from jax.experimental.pallas import tpu as pltpu
from jax.experimental.pallas import tpu_sc as plsc

N = 10000
E = 320000
F = 128
G = 64
EPS = 1e-10
SCALAR = 20

NC = 2
NS = 16
NW = NC * NS
EPW = E // NW
CHUNK = 80
NCHUNK = EPW // CHUNK
ZR = 632
NP = NS * ZR

_mesh = plsc.VectorSubcoreMesh(core_axis_name="c", subcore_axis_name="s")


@functools.partial(
    pl.kernel,
    out_type=jax.ShapeDtypeStruct((NC, NP, F), jnp.float32),
    mesh=_mesh,
    scratch_types=[
        pltpu.VMEM((EPW,), jnp.int32),
        pltpu.VMEM((CHUNK,), jnp.int32),
        pltpu.VMEM((CHUNK,), jnp.int32),
        pltpu.VMEM((CHUNK,), jnp.int32),
        pltpu.VMEM((CHUNK,), jnp.int32),
        pltpu.VMEM((CHUNK, F), jnp.float32),
        pltpu.VMEM((CHUNK, F), jnp.float32),
        pltpu.VMEM_SHARED((NP, F), jnp.float32),
        pltpu.SemaphoreType.DMA,
        pltpu.SemaphoreType.DMA,
        pltpu.SemaphoreType.DMA,
        pltpu.SemaphoreType.DMA,
    ],
)
def _seg_sum(h_hbm, packed_hbm, zeros_hbm, out_hbm,
             packed_v, src0_v, src1_v, dst0_v, dst1_v, rows0_v, rows1_v,
             agg_sh, sem_g0, sem_g1, sem_s0, sem_s1):
    c = lax.axis_index("c")
    s = lax.axis_index("s")
    wid = c * NS + s

    bufs = (rows0_v, rows1_v)
    srcs = (src0_v, src1_v)
    dsts = (dst0_v, dst1_v)
    gsems = (sem_g0, sem_g1)
    ssems = (sem_s0, sem_s1)

    def unpack(i, b):
        for k in range(CHUNK // 16):
            v = packed_v[pl.ds(i * CHUNK + k * 16, 16)]
            srcs[b][pl.ds(k * 16, 16)] = jnp.bitwise_and(v, 0x3FFF)
            dsts[b][pl.ds(k * 16, 16)] = jnp.right_shift(v, 14)

    def g_start(i, b):
        pltpu.async_copy(h_hbm.at[srcs[b]], bufs[b], gsems[b])

    def g_wait(b):
        pltpu.make_async_copy(h_hbm.at[pl.ds(0, CHUNK)], bufs[b],
                              gsems[b]).wait()

    def s_start(i, b):
        pltpu.async_copy(bufs[b], agg_sh.at[dsts[b]], ssems[b], add=True)

    def s_wait(b):
        pltpu.make_async_copy(bufs[b], agg_sh.at[pl.ds(0, CHUNK)],
                              ssems[b]).wait()

    pltpu.sync_copy(packed_hbm.at[wid], packed_v)
    pltpu.sync_copy(zeros_hbm, agg_sh.at[pl.ds(s * ZR, ZR)])
    plsc.subcore_barrier()

    unpack(0, 0)
    g_start(0, 0)
    unpack(1, 1)
    g_start(1, 1)

    def body(j, carry):
        i0 = 2 * j
        g_wait(0)
        s_start(i0, 0)
        s_wait(0)
        unpack(i0 + 2, 0)
        g_start(i0 + 2, 0)
        g_wait(1)
        s_start(i0 + 1, 1)
        s_wait(1)

        @pl.when(j < (NCHUNK - 1) // 2 - 1)
        def _():
            unpack(i0 + 3, 1)
            g_start(i0 + 3, 1)

        return carry

    lax.fori_loop(0, (NCHUNK - 1) // 2, body, 0)
    g_wait(0)
    s_start(NCHUNK - 1, 0)
    s_wait(0)

    plsc.subcore_barrier()
    pltpu.sync_copy(agg_sh.at[pl.ds(s * ZR, ZR)],
                    out_hbm.at[c, pl.ds(s * ZR, ZR)])


def _prescale_body(x_ref, batch_ref, nimp_ref, h0_ref):
    b = batch_ref[...]
    nimp = nimp_ref[...]
    gids = lax.broadcasted_iota(jnp.int32, (N, G), 1)
    mask = b == gids
    neg = jnp.float32(-1e30)
    segmax = jnp.max(jnp.where(mask, nimp, neg), axis=0, keepdims=True)
    pmax = jnp.max(jnp.where(mask, segmax, neg), axis=1, keepdims=True)
    ni = nimp / (pmax + EPS)
    ni = (2.0 * ni - 1.0) / (2.0 * SCALAR) + 1.0
    h0_ref[...] = x_ref[...] * ni


def _prescale(x, batch2d, nimp2d):
    return pl.pallas_call(
        _prescale_body,
        out_shape=jax.ShapeDtypeStruct((N, F), jnp.float32),
    )(x, batch2d, nimp2d)


_RB = 1000


def _layer_body(p_ref, h_ref, W1_ref, b1_ref, W2_ref, b2_ref, o_ref):
    z = p_ref[0] + p_ref[1] + h_ref[...]
    a = jnp.dot(z, W1_ref[...], preferred_element_type=jnp.float32)
    a = jnp.maximum(a + b1_ref[...], 0.0)
    o = jnp.dot(a, W2_ref[...], preferred_element_type=jnp.float32)
    o_ref[...] = jnp.maximum(o + b2_ref[...], 0.0)


def _gin_layer(p, h, W1, b1, W2, b2):
    nblk = N // _RB
    return pl.pallas_call(
        _layer_body,
        grid=(nblk,),
        in_specs=[
            pl.BlockSpec((NC, _RB, F), lambda i: (0, i, 0)),
            pl.BlockSpec((_RB, F), lambda i: (i, 0)),
            pl.BlockSpec((F, F), lambda i: (0, 0)),
            pl.BlockSpec((1, F), lambda i: (0, 0)),
            pl.BlockSpec((F, F), lambda i: (0, 0)),
            pl.BlockSpec((1, F), lambda i: (0, 0)),
        ],
        out_specs=pl.BlockSpec((_RB, F), lambda i: (i, 0)),
        out_shape=jax.ShapeDtypeStruct((N, F), jnp.float32),
    )(p, h, W1, b1, W2, b2)


def _pool_body(x1_ref, x2_ref, x3_ref, batch_ref, o_ref):
    b = batch_ref[...]
    gids = lax.broadcasted_iota(jnp.int32, (N, G), 1)
    mask = (b == gids).astype(jnp.float32)
    counts = jnp.maximum(jnp.sum(mask, axis=0, keepdims=True), 1.0)
    dn = (((0,), (0,)), ((), ()))
    for k, xr in enumerate((x1_ref, x2_ref, x3_ref)):
        p = lax.dot_general(mask, xr[...], dn,
                            preferred_element_type=jnp.float32)
        o_ref[:, k * F:(k + 1) * F] = p / counts.T


def _pool(x1, x2, x3, batch2d):
    return pl.pallas_call(
        _pool_body,
        out_shape=jax.ShapeDtypeStruct((G, 3 * F), jnp.float32),
    )(x1, x2, x3, batch2d)


def kernel(x, edge_index, batch, node_imp,
           W1_0, b1_0, W2_0, b2_0,
           W1_1, b1_1, W2_1, b2_1,
           W1_2, b1_2, W2_2, b2_2):
    src = edge_index[0].astype(jnp.int32)
    dst = edge_index[1].astype(jnp.int32)
    packed = (src | (dst << 14)).reshape(NW, EPW)
    batch2d = batch.astype(jnp.int32).reshape(N, 1)
    nimp2d = node_imp.reshape(N, 1)
    zeros = jnp.zeros((ZR, F), jnp.float32)

    h = _prescale(x, batch2d, nimp2d)

    params = [(W1_0, b1_0, W2_0, b2_0),
              (W1_1, b1_1, W2_1, b2_1),
              (W1_2, b1_2, W2_2, b2_2)]
    xs = []
    for (W1, b1, W2, b2) in params:
        p = _seg_sum(h, packed, zeros)[:, :N]
        h = _gin_layer(p, h, W1, b1.reshape(1, F), W2, b2.reshape(1, F))
        xs.append(h)

    graph_emb = _pool(xs[0], xs[1], xs[2], batch2d)
    node_emb = jnp.concatenate(xs, axis=1)
    return graph_emb, node_emb


# speedup vs baseline: 9.7768x; 1.0193x over previous
import functools

import jax
import jax.numpy as jnp
from jax import lax
from jax.experimental import pallas as pl
from jax.experimental.pallas import tpu as pltpu
from jax.experimental.pallas import tpu_sc as plsc

N = 10000
E = 320000
F = 128
G = 64
EPS = 1e-10
SCALAR = 20

NC = 2
NS = 16
NW = NC * NS
EPW = E // NW
CHUNK = 80
NCHUNK = EPW // CHUNK
ZR = 632
NP = NS * ZR

_mesh = plsc.VectorSubcoreMesh(core_axis_name="c", subcore_axis_name="s")


@functools.partial(
    pl.kernel,
    out_type=jax.ShapeDtypeStruct((NC, NP, F), jnp.float32),
    mesh=_mesh,
    scratch_types=[
        pltpu.VMEM((EPW,), jnp.int32),
        pltpu.VMEM((CHUNK,), jnp.int32),
        pltpu.VMEM((CHUNK,), jnp.int32),
        pltpu.VMEM((CHUNK,), jnp.int32),
        pltpu.VMEM((CHUNK,), jnp.int32),
        pltpu.VMEM((CHUNK, F), jnp.float32),
        pltpu.VMEM((CHUNK, F), jnp.float32),
        pltpu.VMEM_SHARED((NP, F), jnp.float32),
        pltpu.SemaphoreType.DMA,
        pltpu.SemaphoreType.DMA,
        pltpu.SemaphoreType.DMA,
        pltpu.SemaphoreType.DMA,
    ],
)
def _seg_sum(h_hbm, packed_hbm, zeros_hbm, out_hbm,
             packed_v, src0_v, src1_v, dst0_v, dst1_v, rows0_v, rows1_v,
             agg_sh, sem_g0, sem_g1, sem_s0, sem_s1):
    c = lax.axis_index("c")
    s = lax.axis_index("s")
    wid = c * NS + s

    bufs = (rows0_v, rows1_v)
    srcs = (src0_v, src1_v)
    dsts = (dst0_v, dst1_v)
    gsems = (sem_g0, sem_g1)
    ssems = (sem_s0, sem_s1)

    def unpack(i, b):
        for k in range(CHUNK // 16):
            v = packed_v[pl.ds(i * CHUNK + k * 16, 16)]
            srcs[b][pl.ds(k * 16, 16)] = jnp.bitwise_and(v, 0x3FFF)
            dsts[b][pl.ds(k * 16, 16)] = jnp.right_shift(v, 14)

    def g_start(i, b):
        pltpu.async_copy(h_hbm.at[srcs[b]], bufs[b], gsems[b])

    def g_wait(b):
        pltpu.make_async_copy(h_hbm.at[pl.ds(0, CHUNK)], bufs[b],
                              gsems[b]).wait()

    def s_start(i, b):
        pltpu.async_copy(bufs[b], agg_sh.at[dsts[b]], ssems[b], add=True)

    def s_wait(b):
        pltpu.make_async_copy(bufs[b], agg_sh.at[pl.ds(0, CHUNK)],
                              ssems[b]).wait()

    pltpu.sync_copy(packed_hbm.at[wid], packed_v)
    unpack(0, 0)
    g_start(0, 0)
    unpack(1, 1)
    g_start(1, 1)
    pltpu.sync_copy(zeros_hbm, agg_sh.at[pl.ds(s * ZR, ZR)])
    plsc.subcore_barrier()

    def body(j, carry):
        i0 = 2 * j
        g_wait(0)
        s_start(i0, 0)
        s_wait(0)
        unpack(i0 + 2, 0)
        g_start(i0 + 2, 0)
        g_wait(1)
        s_start(i0 + 1, 1)
        s_wait(1)

        @pl.when(j < (NCHUNK - 1) // 2 - 1)
        def _():
            unpack(i0 + 3, 1)
            g_start(i0 + 3, 1)

        return carry

    lax.fori_loop(0, (NCHUNK - 1) // 2, body, 0)
    g_wait(0)
    s_start(NCHUNK - 1, 0)
    s_wait(0)

    plsc.subcore_barrier()
    pltpu.sync_copy(agg_sh.at[pl.ds(s * ZR, ZR)],
                    out_hbm.at[c, pl.ds(s * ZR, ZR)])


def _prescale_body(x_ref, batch_ref, nimp_ref, h0_ref):
    b = batch_ref[...]
    nimp = nimp_ref[...]
    gids = lax.broadcasted_iota(jnp.int32, (N, G), 1)
    mask = b == gids
    neg = jnp.float32(-1e30)
    segmax = jnp.max(jnp.where(mask, nimp, neg), axis=0, keepdims=True)
    pmax = jnp.max(jnp.where(mask, segmax, neg), axis=1, keepdims=True)
    ni = nimp / (pmax + EPS)
    ni = (2.0 * ni - 1.0) / (2.0 * SCALAR) + 1.0
    h0_ref[pl.ds(0, N), :] = x_ref[...] * ni
    h0_ref[pl.ds(N, NP - N), :] = jnp.zeros((NP - N, F), jnp.float32)


def _prescale(x, batch2d, nimp2d):
    return pl.pallas_call(
        _prescale_body,
        out_shape=jax.ShapeDtypeStruct((NP, F), jnp.float32),
    )(x, batch2d, nimp2d)


_RB = 632


def _layer_body(p_ref, h_ref, W1_ref, b1_ref, W2_ref, b2_ref, o_ref):
    z = p_ref[0] + p_ref[1] + h_ref[...]
    a = jnp.dot(z, W1_ref[...], preferred_element_type=jnp.float32)
    a = jnp.maximum(a + b1_ref[...], 0.0)
    o = jnp.dot(a, W2_ref[...], preferred_element_type=jnp.float32)
    o_ref[...] = jnp.maximum(o + b2_ref[...], 0.0)


def _gin_layer(p, h, W1, b1, W2, b2):
    nblk = NP // _RB
    return pl.pallas_call(
        _layer_body,
        grid=(nblk,),
        in_specs=[
            pl.BlockSpec((NC, _RB, F), lambda i: (0, i, 0)),
            pl.BlockSpec((_RB, F), lambda i: (i, 0)),
            pl.BlockSpec((F, F), lambda i: (0, 0)),
            pl.BlockSpec((1, F), lambda i: (0, 0)),
            pl.BlockSpec((F, F), lambda i: (0, 0)),
            pl.BlockSpec((1, F), lambda i: (0, 0)),
        ],
        out_specs=pl.BlockSpec((_RB, F), lambda i: (i, 0)),
        out_shape=jax.ShapeDtypeStruct((NP, F), jnp.float32),
    )(p, h, W1, b1, W2, b2)


def _pool_body(x1_ref, x2_ref, x3_ref, batch_ref, o_ref):
    b = batch_ref[...]
    gids = lax.broadcasted_iota(jnp.int32, (NP, G), 1)
    mask = (b == gids).astype(jnp.float32)
    counts = jnp.maximum(jnp.sum(mask, axis=0, keepdims=True), 1.0)
    dn = (((0,), (0,)), ((), ()))
    for k, xr in enumerate((x1_ref, x2_ref, x3_ref)):
        p = lax.dot_general(mask, xr[...], dn,
                            preferred_element_type=jnp.float32)
        o_ref[:, k * F:(k + 1) * F] = p / counts.T


def _pool(x1, x2, x3, batch2d):
    return pl.pallas_call(
        _pool_body,
        out_shape=jax.ShapeDtypeStruct((G, 3 * F), jnp.float32),
    )(x1, x2, x3, batch2d)


def kernel(x, edge_index, batch, node_imp,
           W1_0, b1_0, W2_0, b2_0,
           W1_1, b1_1, W2_1, b2_1,
           W1_2, b1_2, W2_2, b2_2):
    src = edge_index[0].astype(jnp.int32)
    dst = edge_index[1].astype(jnp.int32)
    packed = (src | (dst << 14)).reshape(NW, EPW)
    batch2d = batch.astype(jnp.int32).reshape(N, 1)
    batch2d_p = jnp.concatenate(
        [batch2d, jnp.full((NP - N, 1), -1, jnp.int32)], axis=0)
    nimp2d = node_imp.reshape(N, 1)
    zeros = jnp.zeros((ZR, F), jnp.float32)

    h = _prescale(x, batch2d, nimp2d)

    params = [(W1_0, b1_0, W2_0, b2_0),
              (W1_1, b1_1, W2_1, b2_1),
              (W1_2, b1_2, W2_2, b2_2)]
    xs = []
    for (W1, b1, W2, b2) in params:
        p = _seg_sum(h, packed, zeros)
        h = _gin_layer(p, h, W1, b1.reshape(1, F), W2, b2.reshape(1, F))
        xs.append(h)

    graph_emb = _pool(xs[0], xs[1], xs[2], batch2d_p)
    node_emb = jnp.concatenate([xx[:N] for xx in xs], axis=1)
    return graph_emb, node_emb


# speedup vs baseline: 10.5787x; 1.0820x over previous
import functools

import jax
import jax.numpy as jnp
from jax import lax
from jax.experimental import pallas as pl
from jax.experimental.pallas import tpu as pltpu
from jax.experimental.pallas import tpu_sc as plsc

N = 10000
E = 320000
F = 128
G = 64
EPS = 1e-10
SCALAR = 20

NC = 2
NS = 16
NW = NC * NS
CHUNK = 128
NCHT = E // CHUNK
NCH = NCHT // NW
NEXTRA = NCHT - NCH * NW
CAP = (NCH + 1) * CHUNK
ZR = 632
NP = NS * ZR

_mesh = plsc.VectorSubcoreMesh(core_axis_name="c", subcore_axis_name="s")


@functools.partial(
    pl.kernel,
    out_type=jax.ShapeDtypeStruct((NC, NP, F), jnp.float32),
    mesh=_mesh,
    scratch_types=[
        pltpu.VMEM((CAP,), jnp.int32),
        pltpu.VMEM((CHUNK,), jnp.int32),
        pltpu.VMEM((CHUNK,), jnp.int32),
        pltpu.VMEM((CHUNK,), jnp.int32),
        pltpu.VMEM((CHUNK,), jnp.int32),
        pltpu.VMEM((CHUNK, F), jnp.float32),
        pltpu.VMEM((CHUNK, F), jnp.float32),
        pltpu.VMEM_SHARED((NP, F), jnp.float32),
        pltpu.SemaphoreType.DMA,
        pltpu.SemaphoreType.DMA,
        pltpu.SemaphoreType.DMA,
        pltpu.SemaphoreType.DMA,
    ],
)
def _seg_sum(h_hbm, packed_hbm, zeros_hbm, out_hbm,
             packed_v, src0_v, src1_v, dst0_v, dst1_v, rows0_v, rows1_v,
             agg_sh, sem_g0, sem_g1, sem_s0, sem_s1):
    c = lax.axis_index("c")
    s = lax.axis_index("s")
    wid = c * NS + s
    base_w = wid * (NCH * CHUNK) + jnp.minimum(wid, NEXTRA) * CHUNK
    has_extra = wid < NEXTRA

    bufs = (rows0_v, rows1_v)
    srcs = (src0_v, src1_v)
    dsts = (dst0_v, dst1_v)
    gsems = (sem_g0, sem_g1)
    ssems = (sem_s0, sem_s1)

    def unpack(i, b):
        for k in range(CHUNK // 16):
            v = packed_v[pl.ds(i * CHUNK + k * 16, 16)]
            srcs[b][pl.ds(k * 16, 16)] = jnp.bitwise_and(v, 0x3FFF)
            dsts[b][pl.ds(k * 16, 16)] = jnp.right_shift(v, 14)

    def g_start(i, b):
        pltpu.async_copy(h_hbm.at[srcs[b]], bufs[b], gsems[b])

    def g_wait(b):
        pltpu.make_async_copy(h_hbm.at[pl.ds(0, CHUNK)], bufs[b],
                              gsems[b]).wait()

    def s_start(i, b):
        pltpu.async_copy(bufs[b], agg_sh.at[dsts[b]], ssems[b], add=True)

    def s_wait(b):
        pltpu.make_async_copy(bufs[b], agg_sh.at[pl.ds(0, CHUNK)],
                              ssems[b]).wait()

    pltpu.sync_copy(packed_hbm.at[pl.ds(base_w, CAP)], packed_v)
    unpack(0, 0)
    g_start(0, 0)
    unpack(1, 1)
    g_start(1, 1)
    pltpu.sync_copy(zeros_hbm, agg_sh.at[pl.ds(s * ZR, ZR)])
    plsc.subcore_barrier()

    def body(j, carry):
        i0 = 2 * j
        g_wait(0)
        s_start(i0, 0)
        s_wait(0)

        @pl.when(jnp.logical_or(i0 + 2 < NCH, has_extra))
        def _():
            unpack(i0 + 2, 0)
            g_start(i0 + 2, 0)

        g_wait(1)
        s_start(i0 + 1, 1)
        s_wait(1)

        @pl.when(i0 + 3 < NCH)
        def _():
            unpack(i0 + 3, 1)
            g_start(i0 + 3, 1)

        return carry

    lax.fori_loop(0, NCH // 2, body, 0)

    @pl.when(has_extra)
    def _():
        g_wait(0)
        s_start(NCH, 0)
        s_wait(0)

    plsc.subcore_barrier()
    pltpu.sync_copy(agg_sh.at[pl.ds(s * ZR, ZR)],
                    out_hbm.at[c, pl.ds(s * ZR, ZR)])


def _prescale_body(x_ref, batch_ref, nimp_ref, h0_ref):
    b = batch_ref[...]
    nimp = nimp_ref[...]
    gids = lax.broadcasted_iota(jnp.int32, (N, G), 1)
    mask = b == gids
    neg = jnp.float32(-1e30)
    segmax = jnp.max(jnp.where(mask, nimp, neg), axis=0, keepdims=True)
    pmax = jnp.max(jnp.where(mask, segmax, neg), axis=1, keepdims=True)
    ni = nimp / (pmax + EPS)
    ni = (2.0 * ni - 1.0) / (2.0 * SCALAR) + 1.0
    h0_ref[pl.ds(0, N), :] = x_ref[...] * ni
    h0_ref[pl.ds(N, NP - N), :] = jnp.zeros((NP - N, F), jnp.float32)


def _prescale(x, batch2d, nimp2d):
    return pl.pallas_call(
        _prescale_body,
        out_shape=jax.ShapeDtypeStruct((NP, F), jnp.float32),
    )(x, batch2d, nimp2d)


_RB = 632


def _layer_body(p_ref, h_ref, W1_ref, b1_ref, W2_ref, b2_ref, o_ref):
    z = p_ref[0] + p_ref[1] + h_ref[...]
    a = jnp.dot(z, W1_ref[...], preferred_element_type=jnp.float32)
    a = jnp.maximum(a + b1_ref[...], 0.0)
    o = jnp.dot(a, W2_ref[...], preferred_element_type=jnp.float32)
    o_ref[...] = jnp.maximum(o + b2_ref[...], 0.0)


def _gin_layer(p, h, W1, b1, W2, b2):
    nblk = NP // _RB
    return pl.pallas_call(
        _layer_body,
        grid=(nblk,),
        in_specs=[
            pl.BlockSpec((NC, _RB, F), lambda i: (0, i, 0)),
            pl.BlockSpec((_RB, F), lambda i: (i, 0)),
            pl.BlockSpec((F, F), lambda i: (0, 0)),
            pl.BlockSpec((1, F), lambda i: (0, 0)),
            pl.BlockSpec((F, F), lambda i: (0, 0)),
            pl.BlockSpec((1, F), lambda i: (0, 0)),
        ],
        out_specs=pl.BlockSpec((_RB, F), lambda i: (i, 0)),
        out_shape=jax.ShapeDtypeStruct((NP, F), jnp.float32),
    )(p, h, W1, b1, W2, b2)


def _pool_body(x1_ref, x2_ref, x3_ref, batch_ref, o_ref):
    b = batch_ref[...]
    gids = lax.broadcasted_iota(jnp.int32, (NP, G), 1)
    mask = (b == gids).astype(jnp.float32)
    counts = jnp.maximum(jnp.sum(mask, axis=0, keepdims=True), 1.0)
    dn = (((0,), (0,)), ((), ()))
    for k, xr in enumerate((x1_ref, x2_ref, x3_ref)):
        p = lax.dot_general(mask, xr[...], dn,
                            preferred_element_type=jnp.float32)
        o_ref[:, k * F:(k + 1) * F] = p / counts.T


def _pool(x1, x2, x3, batch2d):
    return pl.pallas_call(
        _pool_body,
        out_shape=jax.ShapeDtypeStruct((G, 3 * F), jnp.float32),
    )(x1, x2, x3, batch2d)


def kernel(x, edge_index, batch, node_imp,
           W1_0, b1_0, W2_0, b2_0,
           W1_1, b1_1, W2_1, b2_1,
           W1_2, b1_2, W2_2, b2_2):
    src = edge_index[0].astype(jnp.int32)
    dst = edge_index[1].astype(jnp.int32)
    packed = jnp.concatenate(
        [src | (dst << 14), jnp.zeros((CHUNK,), jnp.int32)])
    batch2d = batch.astype(jnp.int32).reshape(N, 1)
    batch2d_p = jnp.concatenate(
        [batch2d, jnp.full((NP - N, 1), -1, jnp.int32)], axis=0)
    nimp2d = node_imp.reshape(N, 1)
    zeros = jnp.zeros((ZR, F), jnp.float32)

    h = _prescale(x, batch2d, nimp2d)

    params = [(W1_0, b1_0, W2_0, b2_0),
              (W1_1, b1_1, W2_1, b2_1),
              (W1_2, b1_2, W2_2, b2_2)]
    xs = []
    for (W1, b1, W2, b2) in params:
        p = _seg_sum(h, packed, zeros)
        h = _gin_layer(p, h, W1, b1.reshape(1, F), W2, b2.reshape(1, F))
        xs.append(h)

    graph_emb = _pool(xs[0], xs[1], xs[2], batch2d_p)
    node_emb = jnp.concatenate([xx[:N] for xx in xs], axis=1)
    return graph_emb, node_emb
